# Initial kernel scaffold; baseline (speedup 1.0000x reference)
import jax
import jax.numpy as jnp
from jax.experimental import pallas as pl


def kernel(edge_index, edges, emb, W1, b1, W2, b2, PW1, Pb1, PW2, Pb2):
    raise NotImplementedError("write your pallas kernel here")


# speedup vs baseline: 15.7461x; 15.7461x over previous
import functools

import jax
import jax.numpy as jnp
from jax import lax
from jax.experimental import pallas as pl
from jax.experimental.pallas import tpu as pltpu
from jax.experimental.pallas import tpu_sc as plsc

NN = 10000
EE = 320000
QQ = 65536
DD = 128

NC = 2
NS = 16
NW = NC * NS

C = 100
CH = EE // (NW * C)
NP = 10240
RPT = NP // NS
RZB = 32

_f32 = jnp.float32


def _zero_rows(ref, nrows, width):
    zv = jnp.zeros((16,), _f32)

    def body(i, _):
        for cidx in range(width // 16):
            ref[i, pl.ds(cidx * 16, 16)] = zv
        return 0

    lax.fori_loop(0, nrows, body, 0)


def _fill_rows(ref, nrows, width, value):
    vv = jnp.full((16,), value, _f32)

    def body(i, _):
        for cidx in range(width // 16):
            ref[i, pl.ds(cidx * 16, 16)] = vv
        return 0

    lax.fori_loop(0, nrows, body, 0)


def _make_segsum_kernel(width, gather):
    mesh = plsc.VectorSubcoreMesh(core_axis_name="c", subcore_axis_name="s")
    scratch = [
        pltpu.VMEM((CH, C), jnp.int32),
        pltpu.VMEM((CH, C), jnp.int32),
        pltpu.VMEM((C, width), _f32),
        pltpu.VMEM((RZB, width), _f32),
        pltpu.VMEM_SHARED((NP, width), _f32),
        pltpu.SemaphoreType.DMA,
    ]
    out_type = jax.ShapeDtypeStruct((NC, NP, width), _f32)

    def body(dst_hbm, src_hbm, z_hbm, out_hbm, dst_v, src_v, buf, zb, acc, sem):
        cid = lax.axis_index("c")
        sid = lax.axis_index("s")
        wid = cid * NS + sid

        _zero_rows(zb, RZB, width)
        for r in range(RPT // RZB):
            pltpu.sync_copy(zb, acc.at[pl.ds(sid * RPT + r * RZB, RZB), :])

        pltpu.sync_copy(dst_hbm.at[wid], dst_v)
        if gather:
            pltpu.sync_copy(src_hbm.at[wid], src_v)
        else:
            _fill_rows(buf, C, width, 1.0)

        plsc.subcore_barrier()

        def chunk(j, _):
            if gather:
                pltpu.async_copy(z_hbm.at[src_v.at[j]], buf, sem).wait()
            pltpu.sync_copy(buf, acc.at[dst_v.at[j]], add=True)
            return 0

        lax.fori_loop(0, CH, chunk, 0)

        plsc.subcore_barrier()

        for r in range(RPT // RZB):
            r0 = sid * RPT + r * RZB
            pltpu.sync_copy(acc.at[pl.ds(r0, RZB), :], zb)
            pltpu.sync_copy(zb, out_hbm.at[cid, pl.ds(r0, RZB), :])

    return pl.kernel(body, out_type=out_type, mesh=mesh, scratch_types=scratch)


def _make_pair_gather_kernel():
    PC = 128
    PCH = 2 * QQ // (NW * PC)
    mesh = plsc.VectorSubcoreMesh(core_axis_name="c", subcore_axis_name="s")
    scratch = [
        pltpu.VMEM((PCH, PC), jnp.int32),
        pltpu.VMEM((PC, DD), _f32),
        pltpu.SemaphoreType.DMA,
    ]
    out_type = jax.ShapeDtypeStruct((2 * QQ, DD), _f32)

    def body(pidx_hbm, x_hbm, out_hbm, idx_v, buf, sem):
        cid = lax.axis_index("c")
        sid = lax.axis_index("s")
        wid = cid * NS + sid
        pltpu.sync_copy(pidx_hbm.at[wid], idx_v)

        def chunk(j, _):
            pltpu.async_copy(x_hbm.at[idx_v.at[j]], buf, sem).wait()
            pltpu.sync_copy(buf, out_hbm.at[pl.ds(wid * PCH * PC + j * PC, PC), :])
            return 0

        lax.fori_loop(0, PCH, chunk, 0)

    return pl.kernel(body, out_type=out_type, mesh=mesh, scratch_types=scratch)


_MB = 1000
_GRID = NN // _MB


def _tc_layer1(degp, emb, W1):
    def body(degp_ref, emb_ref, w_ref, z_ref, dis_ref):
        deg = degp_ref[0, :, 0:1] + degp_ref[1, :, 0:1] + 1.0
        dis = lax.rsqrt(deg)
        z_ref[...] = jnp.dot(emb_ref[...] * dis, w_ref[...],
                             preferred_element_type=_f32)
        dis_ref[...] = dis

    return pl.pallas_call(
        body,
        grid=(_GRID,),
        in_specs=[
            pl.BlockSpec((NC, _MB, 16), lambda i: (0, i, 0)),
            pl.BlockSpec((_MB, DD), lambda i: (i, 0)),
            pl.BlockSpec((DD, DD), lambda i: (0, 0)),
        ],
        out_specs=[
            pl.BlockSpec((_MB, DD), lambda i: (i, 0)),
            pl.BlockSpec((_MB, 1), lambda i: (i, 0)),
        ],
        out_shape=[
            jax.ShapeDtypeStruct((NN, DD), _f32),
            jax.ShapeDtypeStruct((NN, 1), _f32),
        ],
    )(degp, emb, W1)


def _tc_layer2(t1p, z1, dis, b1, W2):
    def body(tp_ref, z_ref, dis_ref, b_ref, w_ref, out_ref):
        t = tp_ref[0] + tp_ref[1] + z_ref[...]
        x1 = jax.nn.relu(dis_ref[...] * t + b_ref[...][None, :])
        out_ref[...] = jnp.dot(dis_ref[...] * x1, w_ref[...],
                               preferred_element_type=_f32)

    return pl.pallas_call(
        body,
        grid=(_GRID,),
        in_specs=[
            pl.BlockSpec((NC, _MB, DD), lambda i: (0, i, 0)),
            pl.BlockSpec((_MB, DD), lambda i: (i, 0)),
            pl.BlockSpec((_MB, 1), lambda i: (i, 0)),
            pl.BlockSpec((DD,), lambda i: (0,)),
            pl.BlockSpec((DD, DD), lambda i: (0, 0)),
        ],
        out_specs=pl.BlockSpec((_MB, DD), lambda i: (i, 0)),
        out_shape=jax.ShapeDtypeStruct((NN, DD), _f32),
    )(t1p, z1, dis, b1, W2)


def _tc_layer3(t2p, z2, dis, b2):
    def body(tp_ref, z_ref, dis_ref, b_ref, out_ref):
        t = tp_ref[0] + tp_ref[1] + z_ref[...]
        out_ref[...] = dis_ref[...] * t + b_ref[...][None, :]

    return pl.pallas_call(
        body,
        grid=(_GRID,),
        in_specs=[
            pl.BlockSpec((NC, _MB, DD), lambda i: (0, i, 0)),
            pl.BlockSpec((_MB, DD), lambda i: (i, 0)),
            pl.BlockSpec((_MB, 1), lambda i: (i, 0)),
            pl.BlockSpec((DD,), lambda i: (0,)),
        ],
        out_specs=pl.BlockSpec((_MB, DD), lambda i: (i, 0)),
        out_shape=jax.ShapeDtypeStruct((NN, DD), _f32),
    )(t2p, z2, dis, b2)


def _tc_predictor(xs, xt, PW1, Pb1, PW2, Pb2):
    QB = 2048

    def body(xs_ref, xt_ref, w1_ref, b1_ref, w2_ref, b2_ref, out_ref):
        h = xs_ref[...] * xt_ref[...]
        a = jax.nn.relu(jnp.dot(h, w1_ref[...], preferred_element_type=_f32)
                        + b1_ref[...][None, :])
        o = jnp.dot(a, w2_ref[...], preferred_element_type=_f32) + b2_ref[...][None, :]
        out_ref[...] = jax.nn.sigmoid(o)

    return pl.pallas_call(
        body,
        grid=(QQ // QB,),
        in_specs=[
            pl.BlockSpec((QB, DD), lambda i: (i, 0)),
            pl.BlockSpec((QB, DD), lambda i: (i, 0)),
            pl.BlockSpec((DD, DD), lambda i: (0, 0)),
            pl.BlockSpec((DD,), lambda i: (0,)),
            pl.BlockSpec((DD, 1), lambda i: (0, 0)),
            pl.BlockSpec((1,), lambda i: (0,)),
        ],
        out_specs=pl.BlockSpec((QB, 1), lambda i: (i, 0)),
        out_shape=jax.ShapeDtypeStruct((QQ, 1), _f32),
    )(xs, xt, PW1, Pb1, PW2, Pb2)


def kernel(edge_index, edges, emb, W1, b1, W2, b2, PW1, Pb1, PW2, Pb2):
    src3 = edge_index[0].reshape(NW, CH, C).astype(jnp.int32)
    dst3 = edge_index[1].reshape(NW, CH, C).astype(jnp.int32)
    pidx = jnp.concatenate([edges[0], edges[1]]).reshape(NW, 32, 128).astype(jnp.int32)

    count_k = _make_segsum_kernel(16, gather=False)
    seg_k = _make_segsum_kernel(DD, gather=True)
    pair_k = _make_pair_gather_kernel()

    dummy16 = jnp.zeros((NN, 16), _f32)
    degp = count_k(dst3, dst3, dummy16)

    z1, dis = _tc_layer1(degp, emb, W1)
    t1p = seg_k(dst3, src3, z1)
    z2 = _tc_layer2(t1p, z1, dis, b1, W2)
    t2p = seg_k(dst3, src3, z2)
    x2 = _tc_layer3(t2p, z2, dis, b2)

    rows = pair_k(pidx, x2)
    out = _tc_predictor(rows[:QQ], rows[QQ:], PW1, Pb1, PW2, Pb2)
    return out.reshape(QQ)


# speedup vs baseline: 18.7608x; 1.1915x over previous
import jax
import jax.numpy as jnp
from jax import lax
from jax.experimental import pallas as pl
from jax.experimental.pallas import tpu as pltpu
from jax.experimental.pallas import tpu_sc as plsc

NN = 10000
EE = 320000
QQ = 65536
DD = 128

NC = 2
NS = 16
NW = NC * NS

C = 40
CH = EE // (NW * C)
NSEG = 5
SCH = CH // NSEG
NBUF = 5
CC = 100
CCH = EE // (NW * CC)
PBUF = 4
NP = 10240
RPT = NP // NS
ZB = 40

_f32 = jnp.float32


def _fill_rows(ref, nrows, width, value):
    vv = jnp.full((16,), value, _f32)

    def body(i, _):
        for cidx in range(width // 16):
            ref[i, pl.ds(cidx * 16, 16)] = vv
        return 0

    lax.fori_loop(0, nrows, body, 0)


def _make_count_kernel(width):
    mesh = plsc.VectorSubcoreMesh(core_axis_name="c", subcore_axis_name="s")
    scratch = [
        pltpu.VMEM((CCH, CC), jnp.int32),
        pltpu.VMEM((CC, width), _f32),
        pltpu.VMEM((ZB, width), _f32),
        pltpu.VMEM_SHARED((NP, width), _f32),
        pltpu.SemaphoreType.DMA,
    ]
    out_type = jax.ShapeDtypeStruct((NC, NP, width), _f32)

    def body(dst_hbm, out_hbm, dst_v, buf, zb, acc, sem):
        cid = lax.axis_index("c")
        sid = lax.axis_index("s")
        wid = cid * NS + sid

        _fill_rows(zb, ZB, width, 0.0)
        for r in range(RPT // ZB):
            pltpu.sync_copy(zb, acc.at[pl.ds(sid * RPT + r * ZB, ZB), :])
        _fill_rows(buf, CC, width, 1.0)
        pltpu.sync_copy(dst_hbm.at[wid], dst_v)
        plsc.subcore_barrier()

        def chunk(j, _):
            pltpu.sync_copy(buf, acc.at[dst_v.at[j]], add=True)
            return 0

        lax.fori_loop(0, CCH, chunk, 0)
        plsc.subcore_barrier()

        for r in range(RPT // ZB):
            r0 = sid * RPT + r * ZB
            pltpu.sync_copy(acc.at[pl.ds(r0, ZB), :], zb)
            pltpu.sync_copy(zb, out_hbm.at[cid, pl.ds(r0, ZB), :])

    return pl.kernel(body, out_type=out_type, mesh=mesh, scratch_types=scratch)


def _make_segsum_kernel(width):
    mesh = plsc.VectorSubcoreMesh(core_axis_name="c", subcore_axis_name="s")
    scratch = [
        pltpu.VMEM((SCH, C), jnp.int32),
        pltpu.VMEM((SCH, C), jnp.int32),
        [pltpu.VMEM((C, width), _f32) for _ in range(NBUF)],
        pltpu.VMEM_SHARED((NP, width), _f32),
        [pltpu.SemaphoreType.DMA for _ in range(NBUF)],
        [pltpu.SemaphoreType.DMA for _ in range(NBUF)],
    ]
    out_type = jax.ShapeDtypeStruct((NC, NP, width), _f32)

    def body(dst_hbm, src_hbm, z_hbm, out_hbm, dst_v, src_v,
             bufs, acc, gs, ss):
        cid = lax.axis_index("c")
        sid = lax.axis_index("s")
        wid = cid * NS + sid

        _fill_rows(bufs[0], ZB, width, 0.0)
        for r in range(RPT // ZB):
            pltpu.sync_copy(bufs[0].at[pl.ds(0, ZB), :],
                            acc.at[pl.ds(sid * RPT + r * ZB, ZB), :])
        plsc.subcore_barrier()

        for seg in range(NSEG):
            pltpu.sync_copy(dst_hbm.at[wid, seg], dst_v)
            pltpu.sync_copy(src_hbm.at[wid, seg], src_v)

            for b in range(NBUF):
                pltpu.async_copy(z_hbm.at[src_v.at[b]], bufs[b], gs[b])

            def step(t, _):
                j0 = NBUF * t
                for b in range(NBUF):
                    jj = j0 + b
                    pltpu.make_async_copy(z_hbm.at[src_v.at[jj]], bufs[b], gs[b]).wait()
                    pltpu.async_copy(bufs[b], acc.at[dst_v.at[jj]], ss[b], add=True)
                for b in range(NBUF):
                    jj = j0 + b
                    pltpu.make_async_copy(bufs[b], acc.at[dst_v.at[jj]], ss[b]).wait()

                    @pl.when(jj + NBUF < SCH)
                    def _prefetch(b=b, jj=jj):
                        pltpu.async_copy(z_hbm.at[src_v.at[jj + NBUF]], bufs[b], gs[b])

                return 0

            lax.fori_loop(0, SCH // NBUF, step, 0)

        plsc.subcore_barrier()
        pltpu.sync_copy(acc.at[pl.ds(sid * RPT, RPT), :],
                        out_hbm.at[cid, pl.ds(sid * RPT, RPT), :])

    return pl.kernel(body, out_type=out_type, mesh=mesh, scratch_types=scratch)


def _make_pair_gather_kernel():
    PC = 128
    PCH = 2 * QQ // (NW * PC)
    mesh = plsc.VectorSubcoreMesh(core_axis_name="c", subcore_axis_name="s")
    scratch = [
        pltpu.VMEM((PCH, PC), jnp.int32),
        [pltpu.VMEM((PC, DD), _f32) for _ in range(PBUF)],
        [pltpu.SemaphoreType.DMA for _ in range(PBUF)],
        [pltpu.SemaphoreType.DMA for _ in range(PBUF)],
    ]
    out_type = jax.ShapeDtypeStruct((2 * QQ, DD), _f32)

    def body(pidx_hbm, x_hbm, out_hbm, idx_v, bufs, gs, ss):
        cid = lax.axis_index("c")
        sid = lax.axis_index("s")
        wid = cid * NS + sid
        base = wid * PCH * PC

        pltpu.sync_copy(pidx_hbm.at[wid], idx_v)

        for b in range(PBUF):
            pltpu.async_copy(x_hbm.at[idx_v.at[b]], bufs[b], gs[b])

        def step(t, _):
            j0 = PBUF * t
            for b in range(PBUF):
                jj = j0 + b
                pltpu.make_async_copy(x_hbm.at[idx_v.at[jj]], bufs[b], gs[b]).wait()
                pltpu.async_copy(bufs[b], out_hbm.at[pl.ds(base + jj * PC, PC), :], ss[b])
            for b in range(PBUF):
                jj = j0 + b
                pltpu.make_async_copy(bufs[b], out_hbm.at[pl.ds(base + jj * PC, PC), :],
                                      ss[b]).wait()

                @pl.when(jj + PBUF < PCH)
                def _prefetch(b=b, jj=jj):
                    pltpu.async_copy(x_hbm.at[idx_v.at[jj + PBUF]], bufs[b], gs[b])

            return 0

        lax.fori_loop(0, PCH // PBUF, step, 0)

    return pl.kernel(body, out_type=out_type, mesh=mesh, scratch_types=scratch)


_MB = 1000
_GRID = NN // _MB


def _tc_layer1(degp, emb, W1):
    def body(degp_ref, emb_ref, w_ref, z_ref, dis_ref):
        deg = degp_ref[0, :, 0:1] + degp_ref[1, :, 0:1] + 1.0
        dis = lax.rsqrt(deg)
        z_ref[...] = jnp.dot(emb_ref[...] * dis, w_ref[...],
                             preferred_element_type=_f32)
        dis_ref[...] = dis

    return pl.pallas_call(
        body,
        grid=(_GRID,),
        in_specs=[
            pl.BlockSpec((NC, _MB, DD), lambda i: (0, i, 0)),
            pl.BlockSpec((_MB, DD), lambda i: (i, 0)),
            pl.BlockSpec((DD, DD), lambda i: (0, 0)),
        ],
        out_specs=[
            pl.BlockSpec((_MB, DD), lambda i: (i, 0)),
            pl.BlockSpec((_MB, 1), lambda i: (i, 0)),
        ],
        out_shape=[
            jax.ShapeDtypeStruct((NN, DD), _f32),
            jax.ShapeDtypeStruct((NN, 1), _f32),
        ],
    )(degp, emb, W1)


def _tc_layer2(t1p, z1, dis, b1, W2):
    def body(tp_ref, z_ref, dis_ref, b_ref, w_ref, out_ref):
        t = tp_ref[0] + tp_ref[1] + z_ref[...]
        x1 = jax.nn.relu(dis_ref[...] * t + b_ref[...][None, :])
        out_ref[...] = jnp.dot(dis_ref[...] * x1, w_ref[...],
                               preferred_element_type=_f32)

    return pl.pallas_call(
        body,
        grid=(_GRID,),
        in_specs=[
            pl.BlockSpec((NC, _MB, DD), lambda i: (0, i, 0)),
            pl.BlockSpec((_MB, DD), lambda i: (i, 0)),
            pl.BlockSpec((_MB, 1), lambda i: (i, 0)),
            pl.BlockSpec((DD,), lambda i: (0,)),
            pl.BlockSpec((DD, DD), lambda i: (0, 0)),
        ],
        out_specs=pl.BlockSpec((_MB, DD), lambda i: (i, 0)),
        out_shape=jax.ShapeDtypeStruct((NN, DD), _f32),
    )(t1p, z1, dis, b1, W2)


def _tc_layer3(t2p, z2, dis, b2):
    def body(tp_ref, z_ref, dis_ref, b_ref, out_ref):
        t = tp_ref[0] + tp_ref[1] + z_ref[...]
        out_ref[...] = dis_ref[...] * t + b_ref[...][None, :]

    return pl.pallas_call(
        body,
        grid=(_GRID,),
        in_specs=[
            pl.BlockSpec((NC, _MB, DD), lambda i: (0, i, 0)),
            pl.BlockSpec((_MB, DD), lambda i: (i, 0)),
            pl.BlockSpec((_MB, 1), lambda i: (i, 0)),
            pl.BlockSpec((DD,), lambda i: (0,)),
        ],
        out_specs=pl.BlockSpec((_MB, DD), lambda i: (i, 0)),
        out_shape=jax.ShapeDtypeStruct((NN, DD), _f32),
    )(t2p, z2, dis, b2)


def _tc_predictor(rows, PW1, Pb1, PW2, Pb2):
    QB = 2048
    NQB = QQ // QB

    def body(xs_ref, xt_ref, w1_ref, b1_ref, w2_ref, b2_ref, out_ref):
        h = xs_ref[...] * xt_ref[...]
        a = jax.nn.relu(jnp.dot(h, w1_ref[...], preferred_element_type=_f32)
                        + b1_ref[...][None, :])
        o = jnp.dot(a, w2_ref[...], preferred_element_type=_f32) + b2_ref[...][None, :]
        out_ref[...] = jax.nn.sigmoid(o)

    return pl.pallas_call(
        body,
        grid=(NQB,),
        in_specs=[
            pl.BlockSpec((QB, DD), lambda i: (i, 0)),
            pl.BlockSpec((QB, DD), lambda i: (i + NQB, 0)),
            pl.BlockSpec((DD, DD), lambda i: (0, 0)),
            pl.BlockSpec((DD,), lambda i: (0,)),
            pl.BlockSpec((DD, 1), lambda i: (0, 0)),
            pl.BlockSpec((1,), lambda i: (0,)),
        ],
        out_specs=pl.BlockSpec((QB, 1), lambda i: (i, 0)),
        out_shape=jax.ShapeDtypeStruct((QQ, 1), _f32),
    )(rows, rows, PW1, Pb1, PW2, Pb2)


def kernel(edge_index, edges, emb, W1, b1, W2, b2, PW1, Pb1, PW2, Pb2):
    src4 = edge_index[0].reshape(NW, NSEG, SCH, C).astype(jnp.int32)
    dst4 = edge_index[1].reshape(NW, NSEG, SCH, C).astype(jnp.int32)
    src3 = src4.reshape(NW, CH, C)
    dst3 = dst4.reshape(NW, CH, C)
    pidx = jnp.concatenate([edges[0], edges[1]]).reshape(NW, 32, 128).astype(jnp.int32)

    seg_k = _make_segsum_kernel(DD)
    pair_k = _make_pair_gather_kernel()

    ones = jnp.ones((NN, DD), _f32)
    degp = seg_k(dst4, dst4, ones)

    z1, dis = _tc_layer1(degp, emb, W1)
    t1p = seg_k(dst4, src4, z1)
    z2 = _tc_layer2(t1p, z1, dis, b1, W2)
    t2p = seg_k(dst4, src4, z2)
    x2 = _tc_layer3(t2p, z2, dis, b2)

    rows = pair_k(pidx, x2)
    out = _tc_predictor(rows, PW1, Pb1, PW2, Pb2)
    return out.reshape(QQ)
